# baseline (speedup 1.0000x reference)
import functools
import math

import jax
import jax.numpy as jnp
import numpy as np
from jax.experimental import pallas as pl
from jax.experimental.pallas import tpu as pltpu

_BK = 1024


def _fused_kernel(fe_ref, tp_ref, w1a_ref, w1b_ref, b1_ref, div_ref,
                  ws_ref, wc_ref, w2t_ref, wmbt_ref, b2_ref, bm_ref,
                  out_ref, flag_scr, carry_ref, *, max_len, k_total):
    kb = pl.program_id(1)

    @pl.when(kb < 4)
    def _boundary():
        fe = fe_ref[0]
        prev = jnp.where(kb == 0, fe[:1], carry_ref[...])
        shifted = jnp.concatenate([prev, fe[:-1]], axis=0)
        prod = (shifted * fe).astype(jnp.bfloat16)
        ones = jnp.ones((fe.shape[1], 128), jnp.bfloat16)
        sims = jnp.dot(prod, ones, preferred_element_type=jnp.float32)[:, :1]
        flag = sims < 0.7
        r = jax.lax.broadcasted_iota(jnp.int32, (fe.shape[0], 1), 0)
        first = jnp.logical_and(kb == 0, r == 0)
        last = jnp.logical_and(kb == 3, r == fe.shape[0] - 1)
        flag = jnp.logical_or(jnp.logical_or(flag, first), last)
        flag_scr[pl.ds(kb * _BK, _BK), :] = flag.astype(jnp.float32)
        carry_ref[...] = fe[-1:]

    @pl.when(kb == 4)
    def _embed():
        K = k_total
        f = (flag_scr[...].T > 0.5)
        idx = jax.lax.broadcasted_iota(jnp.int32, (1, K), 1)

        start = jnp.where(f, idx, -1)
        s = 1
        while s < K:
            sh = jnp.concatenate(
                [jnp.full((1, s), -1, jnp.int32), start[:, :-s]], axis=1)
            start = jnp.maximum(start, sh)
            s *= 2

        endc = jnp.where(f, idx, K)
        y = jnp.concatenate(
            [endc[:, 1:], jnp.full((1, 1), K, jnp.int32)], axis=1)
        s = 1
        while s < K:
            sh = jnp.concatenate(
                [y[:, s:], jnp.full((1, s), K, jnp.int32)], axis=1)
            y = jnp.minimum(y, sh)
            s *= 2
        end = jnp.minimum(y, K - 1)

        ln = jnp.maximum(end - start, 1).astype(jnp.float32)
        prog = ((idx - start).astype(jnp.float32) / ln).T
        dist = ((end - idx).astype(jnp.float32) / ln).T

        x1 = prog * w1a_ref[...] + dist * w1b_ref[...] + b1_ref[...]
        h = 0.5 * x1 * (1.0 + jax.lax.erf(
            x1 * np.float32(1.0 / math.sqrt(2.0))))

        tp = tp_ref[0].T
        ai = jnp.clip((tp * (max_len - 1)).astype(jnp.int32), 0, max_len - 1)
        ang = ai.astype(jnp.float32) * div_ref[...]

        z = ang * np.float32(0.15915494309189535)
        m = jnp.floor(z + 0.5)
        x = ang - m * np.float32(6.28125)
        x = x - m * np.float32(1.9353071795864769e-03)
        s2 = x * x
        sinx = x * (1.0 + s2 * (np.float32(-1 / 6) + s2 * (np.float32(1 / 120)
                    + s2 * (np.float32(-1 / 5040) + s2 * (np.float32(1 / 362880)
                    + s2 * np.float32(-1 / 39916800))))))
        cosx = 1.0 + s2 * (np.float32(-1 / 2) + s2 * (np.float32(1 / 24)
                    + s2 * (np.float32(-1 / 720) + s2 * (np.float32(1 / 40320)
                    + s2 * (np.float32(-1 / 3628800)
                    + s2 * np.float32(1 / 479001600))))))

        v = jnp.dot(w2t_ref[...], wmbt_ref[...],
                    preferred_element_type=jnp.float32)
        c = jnp.dot(b2_ref[...], wmbt_ref[...],
                    preferred_element_type=jnp.float32) + bm_ref[...]

        feats = jnp.concatenate([sinx, cosx, h], axis=1)
        wsc_v = jnp.concatenate([ws_ref[...], wc_ref[...], v], axis=0)
        out_ref[0] = jnp.dot(feats.astype(jnp.bfloat16),
                             wsc_v.astype(jnp.bfloat16),
                             preferred_element_type=jnp.float32) + c


def kernel(temporal_pos, frame_embs, abs_pe, W1, b1, W2, b2, Wm, bm):
    B, K, D = frame_embs.shape
    max_len, half = abs_pe.shape
    hd = Wm.shape[0]
    nkb = K // _BK

    WmA = Wm[:, :half]
    ws = WmA[:, 0::2].T
    wc = WmA[:, 1::2].T
    w2t = W2.T
    wmbt = Wm[:, half:].T
    w1a = W1[:, 0].reshape(1, half)
    w1b = W1[:, 1].reshape(1, half)
    b1r = b1.reshape(1, half)
    b2r = b2.reshape(1, half)
    bmr = bm.reshape(1, hd)
    div = np.exp(np.arange(0, half, 2, dtype=np.float32)
                 * (-math.log(10000.0) / half)).reshape(1, half // 2)
    div = jnp.asarray(div)

    def full(shape):
        return pl.BlockSpec(shape, lambda *_: (0,) * len(shape))

    out = pl.pallas_call(
        functools.partial(_fused_kernel, max_len=max_len, k_total=K),
        grid=(B, nkb + 1),
        in_specs=[
            pl.BlockSpec((1, _BK, D),
                         lambda b, kb: (b, jnp.minimum(kb, 3), 0)),
            pl.BlockSpec((1, 1, K), lambda b, kb: (b, 0, 0)),
            full((1, half)),
            full((1, half)),
            full((1, half)),
            full((1, half // 2)),
            full((half // 2, hd)),
            full((half // 2, hd)),
            full((half, half)),
            full((half, hd)),
            full((1, half)),
            full((1, hd)),
        ],
        out_specs=pl.BlockSpec((1, K, hd), lambda b, kb: (b, 0, 0)),
        out_shape=jax.ShapeDtypeStruct((B, K, hd), jnp.float32),
        scratch_shapes=[pltpu.VMEM((K, 1), jnp.float32),
                        pltpu.VMEM((1, D), jnp.float32)],
    )(frame_embs, temporal_pos.reshape(B, 1, K),
      w1a, w1b, b1r, div, ws, wc, w2t, wmbt, b2r, bmr)
    return out


# speedup vs baseline: 1.1835x; 1.1835x over previous
import functools
import math

import jax
import jax.numpy as jnp
import numpy as np
from jax.experimental import pallas as pl
from jax.experimental.pallas import tpu as pltpu

_BK = 2048


def _boundary_kernel(fe_ref, flags_ref, carry_ref):
    kb = pl.program_id(1)
    nkb = pl.num_programs(1)
    fe = fe_ref[0]
    prev = carry_ref[...]
    shifted = jnp.concatenate([prev, fe[:-1]], axis=0)
    prod = (shifted * fe).astype(jnp.bfloat16)
    ones = jnp.ones((fe.shape[1], 128), jnp.bfloat16)
    sims = jnp.dot(prod, ones, preferred_element_type=jnp.float32)[:, :1]
    flag = sims < 0.7
    r = jax.lax.broadcasted_iota(jnp.int32, (fe.shape[0], 1), 0)
    first = jnp.logical_and(kb == 0, r == 0)
    last = jnp.logical_and(kb == nkb - 1, r == fe.shape[0] - 1)
    flag = jnp.logical_or(jnp.logical_or(flag, first), last)
    flags_ref[0] = flag.astype(jnp.int32)
    carry_ref[...] = fe[-1:]


def _embed_kernel(flags_ref, tp_ref, w1a_ref, w1b_ref, b1_ref, div_ref,
                  ws_ref, wc_ref, w2t_ref, wmbt_ref, b2_ref, bm_ref,
                  out_ref, prog_scr, dist_scr, *, max_len):
    b = pl.program_id(0)
    K = flags_ref.shape[1]

    @pl.when(b == 0)
    def _scan_all():
        f = flags_ref[...]
        Bn = f.shape[0]
        idx = jax.lax.broadcasted_iota(jnp.int32, (Bn, K), 1)

        start = jnp.where(f > 0, idx, -1)
        s = 1
        while s < K:
            sh = jnp.concatenate(
                [jnp.full((Bn, s), -1, jnp.int32), start[:, :-s]], axis=1)
            start = jnp.maximum(start, sh)
            s *= 2

        endc = jnp.where(f > 0, idx, K)
        y = jnp.concatenate(
            [endc[:, 1:], jnp.full((Bn, 1), K, jnp.int32)], axis=1)
        s = 1
        while s < K:
            sh = jnp.concatenate(
                [y[:, s:], jnp.full((Bn, s), K, jnp.int32)], axis=1)
            y = jnp.minimum(y, sh)
            s *= 2
        end = jnp.minimum(y, K - 1)

        ln = jnp.maximum(end - start, 1).astype(jnp.float32)
        prog_scr[...] = (idx - start).astype(jnp.float32) / ln
        dist_scr[...] = (end - idx).astype(jnp.float32) / ln

    prog = prog_scr[pl.ds(b, 1), :].T
    dist = dist_scr[pl.ds(b, 1), :].T

    x1 = prog * w1a_ref[...] + dist * w1b_ref[...] + b1_ref[...]
    h = 0.5 * x1 * (1.0 + jax.lax.erf(x1 * np.float32(1.0 / math.sqrt(2.0))))

    tp = tp_ref[0].T
    ai = jnp.clip((tp * (max_len - 1)).astype(jnp.int32), 0, max_len - 1)
    ang = ai.astype(jnp.float32) * div_ref[...]

    v = jnp.dot(w2t_ref[...], wmbt_ref[...],
                preferred_element_type=jnp.float32)
    c = jnp.dot(b2_ref[...], wmbt_ref[...],
                preferred_element_type=jnp.float32) + bm_ref[...]

    z = ang * np.float32(0.15915494309189535)
    m = jnp.floor(z + 0.5)
    x = ang - m * np.float32(6.28125)
    x = x - m * np.float32(1.9353071795864769e-03)
    s = x * x
    sinx = x * (1.0 + s * (np.float32(-1 / 6) + s * (np.float32(1 / 120)
                + s * (np.float32(-1 / 5040) + s * (np.float32(1 / 362880)
                + s * np.float32(-1 / 39916800))))))
    cosx = 1.0 + s * (np.float32(-1 / 2) + s * (np.float32(1 / 24)
                + s * (np.float32(-1 / 720) + s * (np.float32(1 / 40320)
                + s * (np.float32(-1 / 3628800)
                + s * np.float32(1 / 479001600))))))
    feats = jnp.concatenate([sinx, cosx, h], axis=1)
    wsc_v = jnp.concatenate([ws_ref[...], wc_ref[...], v], axis=0)
    out_ref[0] = jnp.dot(feats.astype(jnp.bfloat16),
                         wsc_v.astype(jnp.bfloat16),
                         preferred_element_type=jnp.float32) + c


def kernel(temporal_pos, frame_embs, abs_pe, W1, b1, W2, b2, Wm, bm):
    B, K, D = frame_embs.shape
    max_len, half = abs_pe.shape
    hd = Wm.shape[0]
    nkb = K // _BK

    flags = pl.pallas_call(
        _boundary_kernel,
        grid=(B, nkb),
        in_specs=[pl.BlockSpec((1, _BK, D), lambda b, kb: (b, kb, 0))],
        out_specs=pl.BlockSpec((1, _BK, 1), lambda b, kb: (b, kb, 0)),
        out_shape=jax.ShapeDtypeStruct((B, K, 1), jnp.int32),
        scratch_shapes=[pltpu.VMEM((1, D), jnp.float32)],
    )(frame_embs)


    WmA = Wm[:, :half]
    ws = WmA[:, 0::2].T
    wc = WmA[:, 1::2].T
    w2t = W2.T
    wmbt = Wm[:, half:].T
    w1a = W1[:, 0].reshape(1, half)
    w1b = W1[:, 1].reshape(1, half)
    b1r = b1.reshape(1, half)
    b2r = b2.reshape(1, half)
    bmr = bm.reshape(1, hd)
    div = np.exp(np.arange(0, half, 2, dtype=np.float32)
                 * (-math.log(10000.0) / half)).reshape(1, half // 2)
    div = jnp.asarray(div)

    def full(shape):
        return pl.BlockSpec(shape, lambda *_: (0,) * len(shape))

    row = pl.BlockSpec((1, 1, K), lambda b: (b, 0, 0))
    out = pl.pallas_call(
        functools.partial(_embed_kernel, max_len=max_len),
        grid=(B,),
        in_specs=[
            full((B, K)),
            row,
            full((1, half)),
            full((1, half)),
            full((1, half)),
            full((1, half // 2)),
            full((half // 2, hd)),
            full((half // 2, hd)),
            full((half, half)),
            full((half, hd)),
            full((1, half)),
            full((1, hd)),
        ],
        out_specs=pl.BlockSpec((1, K, hd), lambda b: (b, 0, 0)),
        out_shape=jax.ShapeDtypeStruct((B, K, hd), jnp.float32),
        scratch_shapes=[pltpu.VMEM((B, K), jnp.float32),
                        pltpu.VMEM((B, K), jnp.float32)],
    )(flags.reshape(B, K), temporal_pos.reshape(B, 1, K),
      w1a, w1b, b1r, div, ws, wc, w2t, wmbt, b2r, bmr)
    return out
